# baseline (speedup 1.0000x reference)
import functools

import jax
import jax.numpy as jnp
from jax import lax
from jax.experimental import pallas as pl
from jax.experimental.pallas import tpu as pltpu
from jax.experimental.pallas import tpu_sc as plsc

N = 10000
E = 320000
D = 128
H = 8
DH = D // H
AW = D + 32
AF = D + H


NC = 2
NS = 16
NW = NC * NS
EPC = E // NC
EPT = EPC // NS
C = 80
NCH = EPT // C
NP = 10240
RPT = NP // NS


def _prep_body(x_ref, w_ref, asf_ref, adf_ref, h_ref, ts_ref, td_ref,
               init_ref):
    xb = x_ref[...]
    h = jnp.dot(xb, w_ref[...], preferred_element_type=jnp.float32)
    h_ref[...] = h
    b = h.shape[0]
    a_s = (h * asf_ref[...]).reshape(b, H, DH).sum(-1)
    a_d = (h * adf_ref[...]).reshape(b, H, DH).sum(-1)
    ts_ref[...] = jnp.concatenate([a_s, a_s], axis=-1)
    td_ref[...] = jnp.concatenate([a_d, a_d], axis=-1)
    el = a_s + a_d
    el = jnp.maximum(el, 0.2 * el)
    exl = jnp.exp(el)
    exl128 = jnp.broadcast_to(exl[:, :, None], (b, H, DH)).reshape(b, D)
    init_ref[...] = jnp.concatenate([exl128 * h, exl], axis=-1)


def _prep_call(x, W, a_src, a_dst):
    bl = 1000
    grid = (N // bl,)
    return pl.pallas_call(
        _prep_body,
        grid=grid,
        in_specs=[
            pl.BlockSpec((bl, D), lambda i: (i, 0)),
            pl.BlockSpec((D, D), lambda i: (0, 0)),
            pl.BlockSpec((1, D), lambda i: (0, 0)),
            pl.BlockSpec((1, D), lambda i: (0, 0)),
        ],
        out_specs=[
            pl.BlockSpec((bl, D), lambda i: (i, 0)),
            pl.BlockSpec((bl, 2 * H), lambda i: (i, 0)),
            pl.BlockSpec((bl, 2 * H), lambda i: (i, 0)),
            pl.BlockSpec((bl, AF), lambda i: (i, 0)),
        ],
        out_shape=[
            jax.ShapeDtypeStruct((N, D), jnp.float32),
            jax.ShapeDtypeStruct((N, 2 * H), jnp.float32),
            jax.ShapeDtypeStruct((N, 2 * H), jnp.float32),
            jax.ShapeDtypeStruct((N, AF), jnp.float32),
        ],
    )(x, W, a_src.reshape(1, D), a_dst.reshape(1, D))


def _edge_body(zeros_hbm, src3_hbm, dst3_hbm, ts_hbm, td_hbm, h_hbm, out_hbm,
               acc, sall, dall, asb0, adb0, hb0, asb1, adb1, hb1, mb,
               sg0, sg1):
    cid = lax.axis_index("c")
    sid = lax.axis_index("s")
    rbase = sid * RPT
    w = cid * NS + sid
    pltpu.sync_copy(zeros_hbm, acc.at[pl.ds(rbase, RPT)])
    pltpu.sync_copy(src3_hbm.at[w], sall)
    pltpu.sync_copy(dst3_hbm.at[w], dall)
    plsc.subcore_barrier()

    def issue(j, asb, adb, hb, sg):
        pltpu.async_copy(ts_hbm.at[sall.at[j]], asb, sg)
        pltpu.async_copy(td_hbm.at[dall.at[j]], adb, sg)
        pltpu.async_copy(h_hbm.at[sall.at[j]], hb, sg)

    def compute(j, asb, adb, hb, sg):
        pltpu.make_async_copy(ts_hbm.at[sall.at[j]], asb, sg).wait()
        pltpu.make_async_copy(td_hbm.at[dall.at[j]], adb, sg).wait()
        pltpu.make_async_copy(h_hbm.at[sall.at[j]], hb, sg).wait()

        @plsc.parallel_loop(0, C, unroll=8)
        def edge_one(i):
            e = asb[i] + adb[i]
            e = jnp.maximum(e, 0.2 * e)
            ex = jnp.exp(e)
            mb[i, pl.ds(D, 32)] = plsc.pack(
                ex, ex, format=plsc.PackFormat.INTERLEAVED)
            for t in range(4):
                m0 = jnp.full((16,), ex[2 * t], jnp.float32)
                m1 = jnp.full((16,), ex[2 * t + 1], jnp.float32)
                off = 32 * t
                p0 = hb[i, pl.ds(off, 16)] * m0
                p1 = hb[i, pl.ds(off + 16, 16)] * m1
                mb[i, pl.ds(off, 32)] = plsc.pack(
                    p0, p1, format=plsc.PackFormat.INTERLEAVED)

        pltpu.sync_copy(mb, acc.at[dall.at[j]], add=True)

    issue(0, asb0, adb0, hb0, sg0)

    def pair(g, _):
        j0 = 2 * g
        issue(j0 + 1, asb1, adb1, hb1, sg1)
        compute(j0, asb0, adb0, hb0, sg0)
        issue(j0 + 2, asb0, adb0, hb0, sg0)
        compute(j0 + 1, asb1, adb1, hb1, sg1)
        return 0

    lax.fori_loop(0, (NCH - 1) // 2, pair, 0)
    compute(NCH - 1, asb0, adb0, hb0, sg0)

    plsc.subcore_barrier()
    pltpu.sync_copy(acc.at[pl.ds(rbase, RPT)],
                    out_hbm.at[pl.ds(cid * NP + rbase, RPT)])


def _edge_call(edge_index, ts, td, h):
    mesh = plsc.VectorSubcoreMesh(core_axis_name="c", subcore_axis_name="s")
    zeros = jnp.zeros((RPT, AW), jnp.bfloat16)
    kern = pl.kernel(
        _edge_body,
        out_type=jax.ShapeDtypeStruct((NC * NP, AW), jnp.bfloat16),
        mesh=mesh,
        scratch_types=[
            pltpu.VMEM_SHARED((NP, AW), jnp.bfloat16),
            pltpu.VMEM((NCH, C), jnp.int32),
            pltpu.VMEM((NCH, C), jnp.int32),
            pltpu.VMEM((C, 2 * H), jnp.float32),
            pltpu.VMEM((C, 2 * H), jnp.float32),
            pltpu.VMEM((C, D), jnp.float32),
            pltpu.VMEM((C, 2 * H), jnp.float32),
            pltpu.VMEM((C, 2 * H), jnp.float32),
            pltpu.VMEM((C, D), jnp.float32),
            pltpu.VMEM((C, AW), jnp.bfloat16),
            pltpu.SemaphoreType.DMA,
            pltpu.SemaphoreType.DMA,
        ],
        compiler_params=pltpu.CompilerParams(use_tc_tiling_on_sc=False,
                                             needs_layout_passes=False),
    )
    src3 = edge_index[0].reshape(NW, NCH, C)
    dst3 = edge_index[1].reshape(NW, NCH, C)
    return kern(zeros, src3, dst3, ts, td, h)


def _final_body(p0_ref, p1_ref, init_ref, x_ref, bias_ref, gamma_ref,
                beta_ref, y_ref):
    u = p0_ref[...].astype(jnp.float32) + p1_ref[...].astype(jnp.float32)
    b = u.shape[0]
    ud = u.reshape(b, AW // 32, DH, 2).transpose(0, 1, 3, 2).reshape(b, AW)
    init = init_ref[...]
    num = ud[:, :D] + init[:, :D]
    den8 = ud[:, D:D + H] + init[:, D:D + H]
    den = jnp.broadcast_to(den8[:, :, None], (b, H, DH)).reshape(b, D)
    out = num / (den + 1e-16) + bias_ref[...]
    y = out + x_ref[...]
    mean = jnp.mean(y, axis=-1, keepdims=True)
    var = jnp.mean((y - mean) ** 2, axis=-1, keepdims=True)
    y = (y - mean) * lax.rsqrt(var + 1e-5)
    y_ref[...] = y * gamma_ref[...] + beta_ref[...]


def _final_call(p, init, x, bias, gamma, beta):
    bl = 1000
    grid = (N // bl,)
    return pl.pallas_call(
        _final_body,
        grid=grid,
        in_specs=[
            pl.BlockSpec((bl, AW), lambda i: (i, 0)),
            pl.BlockSpec((bl, AW), lambda i: (i, 0)),
            pl.BlockSpec((bl, AF), lambda i: (i, 0)),
            pl.BlockSpec((bl, D), lambda i: (i, 0)),
            pl.BlockSpec((1, D), lambda i: (0, 0)),
            pl.BlockSpec((1, D), lambda i: (0, 0)),
            pl.BlockSpec((1, D), lambda i: (0, 0)),
        ],
        out_specs=pl.BlockSpec((bl, D), lambda i: (i, 0)),
        out_shape=jax.ShapeDtypeStruct((N, D), jnp.float32),
    )(p[:N], p[NP:NP + N], init, x, bias.reshape(1, D), gamma.reshape(1, D),
      beta.reshape(1, D))


def kernel(x, edge_index, W, a_src, a_dst, bias, gamma, beta):
    h, ts, td, init = _prep_call(x, W, a_src, a_dst)
    p = _edge_call(edge_index, ts, td, h)
    return _final_call(p, init, x, bias, gamma, beta)


# speedup vs baseline: 1.4573x; 1.4573x over previous
import functools

import jax
import jax.numpy as jnp
from jax import lax
from jax.experimental import pallas as pl
from jax.experimental.pallas import tpu as pltpu
from jax.experimental.pallas import tpu_sc as plsc

N = 10000
E = 320000
D = 128
H = 8
DH = D // H
AW = D + 32
AF = D + H

_PERM = [0] * AF
for _t in range(4):
    for _k in range(DH):
        _PERM[32 * _t + _k] = 32 * _t + 2 * _k
        _PERM[32 * _t + DH + _k] = 32 * _t + 2 * _k + 1
for _h in range(H):
    _PERM[D + _h] = D + 2 * _h

NC = 2
NS = 16
NW = NC * NS
EPC = E // NC
EPT = EPC // NS
C = 80
NCH = EPT // C
NP = 10240
RPT = NP // NS


def _prep_body(x_ref, w_ref, asf_ref, adf_ref, h_ref, ts_ref, td_ref,
               init_ref):
    xb = x_ref[...]
    h = jnp.dot(xb, w_ref[...], preferred_element_type=jnp.float32)
    h_ref[...] = h
    b = h.shape[0]
    a_s = (h * asf_ref[...]).reshape(b, H, DH).sum(-1)
    a_d = (h * adf_ref[...]).reshape(b, H, DH).sum(-1)
    ts_ref[...] = jnp.concatenate([a_s, a_s], axis=-1)
    td_ref[...] = jnp.concatenate([a_d, a_d], axis=-1)
    el = a_s + a_d
    el = jnp.maximum(el, 0.2 * el)
    exl = jnp.exp(el)
    exl128 = jnp.broadcast_to(exl[:, :, None], (b, H, DH)).reshape(b, D)
    init_ref[...] = jnp.concatenate([exl128 * h, exl], axis=-1)


def _prep_call(x, W, a_src, a_dst):
    bl = 1000
    grid = (N // bl,)
    return pl.pallas_call(
        _prep_body,
        grid=grid,
        in_specs=[
            pl.BlockSpec((bl, D), lambda i: (i, 0)),
            pl.BlockSpec((D, D), lambda i: (0, 0)),
            pl.BlockSpec((1, D), lambda i: (0, 0)),
            pl.BlockSpec((1, D), lambda i: (0, 0)),
        ],
        out_specs=[
            pl.BlockSpec((bl, D), lambda i: (i, 0)),
            pl.BlockSpec((bl, 2 * H), lambda i: (i, 0)),
            pl.BlockSpec((bl, 2 * H), lambda i: (i, 0)),
            pl.BlockSpec((bl, AF), lambda i: (i, 0)),
        ],
        out_shape=[
            jax.ShapeDtypeStruct((N, D), jnp.float32),
            jax.ShapeDtypeStruct((N, 2 * H), jnp.float32),
            jax.ShapeDtypeStruct((N, 2 * H), jnp.float32),
            jax.ShapeDtypeStruct((N, AF), jnp.float32),
        ],
    )(x, W, a_src.reshape(1, D), a_dst.reshape(1, D))


def _edge_body(zeros_hbm, src3_hbm, dst3_hbm, ts_hbm, td_hbm, h_hbm, out_hbm,
               acc, sall, dall, asb0, adb0, hb0, asb1, adb1, hb1, mb,
               sg0, sg1):
    cid = lax.axis_index("c")
    sid = lax.axis_index("s")
    rbase = sid * RPT
    w = cid * NS + sid
    pltpu.sync_copy(zeros_hbm, acc.at[pl.ds(rbase, RPT)])
    pltpu.sync_copy(src3_hbm.at[w], sall)
    pltpu.sync_copy(dst3_hbm.at[w], dall)
    plsc.subcore_barrier()

    def issue(j, asb, adb, hb, sg):
        pltpu.async_copy(ts_hbm.at[sall.at[j]], asb, sg)
        pltpu.async_copy(td_hbm.at[dall.at[j]], adb, sg)
        pltpu.async_copy(h_hbm.at[sall.at[j]], hb, sg)

    def compute(j, asb, adb, hb, sg):
        pltpu.make_async_copy(ts_hbm.at[sall.at[j]], asb, sg).wait()
        pltpu.make_async_copy(td_hbm.at[dall.at[j]], adb, sg).wait()
        pltpu.make_async_copy(h_hbm.at[sall.at[j]], hb, sg).wait()

        @plsc.parallel_loop(0, C, unroll=8)
        def edge_one(i):
            e = asb[i] + adb[i]
            e = jnp.maximum(e, 0.2 * e)
            ex = jnp.exp(e)
            mb[i, pl.ds(D, 32)] = plsc.pack(
                ex, ex, format=plsc.PackFormat.INTERLEAVED)
            for t in range(4):
                m0 = jnp.full((16,), ex[2 * t], jnp.float32)
                m1 = jnp.full((16,), ex[2 * t + 1], jnp.float32)
                off = 32 * t
                p0 = hb[i, pl.ds(off, 16)] * m0
                p1 = hb[i, pl.ds(off + 16, 16)] * m1
                mb[i, pl.ds(off, 32)] = plsc.pack(
                    p0, p1, format=plsc.PackFormat.INTERLEAVED)

        pltpu.sync_copy(mb, acc.at[dall.at[j]], add=True)

    issue(0, asb0, adb0, hb0, sg0)

    def pair(g, _):
        j0 = 2 * g
        issue(j0 + 1, asb1, adb1, hb1, sg1)
        compute(j0, asb0, adb0, hb0, sg0)
        issue(j0 + 2, asb0, adb0, hb0, sg0)
        compute(j0 + 1, asb1, adb1, hb1, sg1)
        return 0

    lax.fori_loop(0, (NCH - 1) // 2, pair, 0)
    compute(NCH - 1, asb0, adb0, hb0, sg0)

    plsc.subcore_barrier()
    pltpu.sync_copy(acc.at[pl.ds(rbase, RPT)],
                    out_hbm.at[pl.ds(cid * NP + rbase, RPT)])


def _edge_call(edge_index, ts, td, h):
    mesh = plsc.VectorSubcoreMesh(core_axis_name="c", subcore_axis_name="s")
    zeros = jnp.zeros((RPT, AW), jnp.bfloat16)
    kern = pl.kernel(
        _edge_body,
        out_type=jax.ShapeDtypeStruct((NC * NP, AW), jnp.bfloat16),
        mesh=mesh,
        scratch_types=[
            pltpu.VMEM_SHARED((NP, AW), jnp.bfloat16),
            pltpu.VMEM((NCH, C), jnp.int32),
            pltpu.VMEM((NCH, C), jnp.int32),
            pltpu.VMEM((C, 2 * H), jnp.float32),
            pltpu.VMEM((C, 2 * H), jnp.float32),
            pltpu.VMEM((C, D), jnp.float32),
            pltpu.VMEM((C, 2 * H), jnp.float32),
            pltpu.VMEM((C, 2 * H), jnp.float32),
            pltpu.VMEM((C, D), jnp.float32),
            pltpu.VMEM((C, AW), jnp.bfloat16),
            pltpu.SemaphoreType.DMA,
            pltpu.SemaphoreType.DMA,
        ],
        compiler_params=pltpu.CompilerParams(use_tc_tiling_on_sc=False,
                                             needs_layout_passes=False),
    )
    src3 = edge_index[0].reshape(NW, NCH, C)
    dst3 = edge_index[1].reshape(NW, NCH, C)
    return kern(zeros, src3, dst3, ts, td, h)


def _final_body(p0_ref, p1_ref, init_ref, x_ref, bias_ref, gamma_ref,
                beta_ref, y_ref):
    u = p0_ref[...] + p1_ref[...] + init_ref[...]
    b = u.shape[0]
    num = u[:, :D]
    den8 = u[:, D:D + H]
    den = jnp.broadcast_to(den8[:, :, None], (b, H, DH)).reshape(b, D)
    out = num / (den + 1e-16) + bias_ref[...]
    y = out + x_ref[...]
    mean = jnp.mean(y, axis=-1, keepdims=True)
    var = jnp.mean((y - mean) ** 2, axis=-1, keepdims=True)
    y = (y - mean) * lax.rsqrt(var + 1e-5)
    y_ref[...] = y * gamma_ref[...] + beta_ref[...]


def _final_call(p, init, x, bias, gamma, beta):
    bl = 1000
    grid = (N // bl,)
    return pl.pallas_call(
        _final_body,
        grid=grid,
        in_specs=[
            pl.BlockSpec((bl, AF), lambda i: (i, 0)),
            pl.BlockSpec((bl, AF), lambda i: (i, 0)),
            pl.BlockSpec((bl, AF), lambda i: (i, 0)),
            pl.BlockSpec((bl, D), lambda i: (i, 0)),
            pl.BlockSpec((1, D), lambda i: (0, 0)),
            pl.BlockSpec((1, D), lambda i: (0, 0)),
            pl.BlockSpec((1, D), lambda i: (0, 0)),
        ],
        out_specs=pl.BlockSpec((bl, D), lambda i: (i, 0)),
        out_shape=jax.ShapeDtypeStruct((N, D), jnp.float32),
    )(p[:N], p[NP:NP + N], init, x, bias.reshape(1, D), gamma.reshape(1, D),
      beta.reshape(1, D))


def kernel(x, edge_index, W, a_src, a_dst, bias, gamma, beta):
    h, ts, td, init = _prep_call(x, W, a_src, a_dst)
    p = _edge_call(edge_index, ts, td, h)
    pf = p.astype(jnp.float32)[:, jnp.array(_PERM, dtype=jnp.int32)]
    return _final_call(pf, init, x, bias, gamma, beta)
